# Initial kernel scaffold; baseline (speedup 1.0000x reference)
import jax
import jax.numpy as jnp
from jax.experimental import pallas as pl


def kernel(features, params):
    raise NotImplementedError("write your pallas kernel here")


# speedup vs baseline: 4.3659x; 4.3659x over previous
import functools

import jax
import jax.numpy as jnp
from jax import lax
from jax.experimental import pallas as pl
from jax.experimental.pallas import tpu as pltpu
from jax.experimental.pallas import tpu_sc as plsc

B = 8
C = 128
C4 = C // 4
C2 = 256
CL = 256
K = 9
N = 2000
NP = 2048
TN = 256
NT = NP // TN
EPS = 1e-5


def _nmask(nt, rows):
    gio = lax.broadcasted_iota(jnp.int32, (rows, 1), 0) + nt * TN
    return jnp.where(gio < N, 1.0, 0.0).astype(jnp.float32)


def _wspec(w):
    return pl.BlockSpec(w.shape, lambda b, nt, _nd=w.ndim: (0,) * _nd)


def _knn_body(xcl_ref, xcf_ref, xxi_ref, xxj_ref, idx_ref):
    b = pl.program_id(0)
    xt = xcl_ref[0]
    xcf = xcf_ref[0]
    xxj = xxj_ref[0]
    xxi = xxi_ref[0].reshape(TN, 1)
    dot = jnp.dot(xt, xcf, preferred_element_type=jnp.float32)
    pd = 2.0 * dot - xxi - xxj
    cio = lax.broadcasted_iota(jnp.int32, (TN, NP), 1)
    pd = jnp.where(cio < N, pd, -1e30)
    sels = []
    for j in range(K):
        m = jnp.max(pd, axis=1, keepdims=True)
        sel = jnp.min(jnp.where(pd >= m, cio, NP), axis=1, keepdims=True)
        sels.append(sel + b * NP)
        pd = jnp.where(cio == sel, -1e30, pd)
    idx_ref[0] = jnp.concatenate(sels, axis=1)


def _knn(xcl, xcf, xx):
    return pl.pallas_call(
        _knn_body,
        grid=(B, NT),
        in_specs=[
            pl.BlockSpec((1, TN, C), lambda b, nt: (b, nt, 0)),
            pl.BlockSpec((1, C, NP), lambda b, nt: (b, 0, 0)),
            pl.BlockSpec((1, 1, TN), lambda b, nt: (b, 0, nt)),
            pl.BlockSpec((1, 1, NP), lambda b, nt: (b, 0, 0)),
        ],
        out_specs=pl.BlockSpec((1, TN, K), lambda b, nt: (b, nt, 0)),
        out_shape=jax.ShapeDtypeStruct((B, NP, K), jnp.int32),
    )(xcl, xcf, xx[:, None, :], xx[:, None, :])


_NROWS = B * NP * K
_CHUNK = 128


def _gather_feat(table, idxg):
    info = plsc.get_sparse_core_info()
    nw = info.num_cores * info.num_subcores
    per_w = _NROWS // nw
    nchunk = per_w // _CHUNK
    mesh = plsc.VectorSubcoreMesh(core_axis_name="c", subcore_axis_name="s")

    @functools.partial(
        pl.kernel, mesh=mesh,
        out_type=jax.ShapeDtypeStruct((_NROWS, C), jnp.float32),
        scratch_types=[
            pltpu.VMEM((_CHUNK,), jnp.int32),
            pltpu.VMEM((_CHUNK, C), jnp.float32),
            pltpu.SemaphoreType.DMA,
        ],
    )
    def k(table_hbm, idx_hbm, out_hbm, idx_v, rows_v, sem):
        wid = lax.axis_index("s") * info.num_cores + lax.axis_index("c")
        base = wid * per_w

        def step(c, carry):
            off = pl.multiple_of(base + c * _CHUNK, 8)
            pltpu.sync_copy(idx_hbm.at[pl.ds(off, _CHUNK)], idx_v)
            pltpu.async_copy(table_hbm.at[idx_v], rows_v, sem).wait()
            pltpu.sync_copy(rows_v, out_hbm.at[pl.ds(off, _CHUNK)])
            return carry

        lax.fori_loop(0, nchunk, step, 0)

    return k(table, idxg)


def _m1_body(xcl_ref, f_ref, lw1T, lb1, rwT, rb, W1T, b1g,
             o1_ref, x1_ref, g1_ref, so1_ref, sg1_ref):
    nt = pl.program_id(1)
    xt = xcl_ref[0]
    f = f_ref[0]
    msk = _nmask(nt, TN)
    cen = jnp.broadcast_to(xt[:, None, :], (TN, K, C))
    cat = jnp.concatenate([cen, cen - f], axis=2)
    catf = cat.reshape(TN * K, C2)

    o1 = jnp.dot(catf, lw1T[...], preferred_element_type=jnp.float32)
    o1 = (o1 + lb1[...]).reshape(TN, K, C2) * msk[:, None]
    o1_ref[0] = o1
    so1_ref[0, 0] = jnp.stack(
        [jnp.sum(o1, axis=(0, 1)), jnp.sum(o1 * o1, axis=(0, 1))])

    x1 = jnp.dot(catf, rwT[...], preferred_element_type=jnp.float32)
    x1_ref[0] = (x1 + rb[...]).reshape(TN, K, C2) * msk[:, None]

    gs = jnp.zeros((C2,), jnp.float32)
    gq = jnp.zeros((C2,), jnp.float32)
    for j in range(3):
        acc = b1g[...] * jnp.ones((TN, 1), jnp.float32)
        for t in range(3):
            acc = acc + jnp.dot(cat[:, 3 * j + t, :], W1T[t],
                                preferred_element_type=jnp.float32)
        acc = acc * msk
        g1_ref[0, :, j] = acc
        gs = gs + jnp.sum(acc, axis=0)
        gq = gq + jnp.sum(acc * acc, axis=0)
    sg1_ref[0, 0] = jnp.stack([gs, gq])


def _m1(xcl, feat, wts):
    return pl.pallas_call(
        _m1_body,
        grid=(B, NT),
        in_specs=[
            pl.BlockSpec((1, TN, C), lambda b, nt: (b, nt, 0)),
            pl.BlockSpec((1, TN, K, C), lambda b, nt: (b, nt, 0, 0)),
        ] + [_wspec(w) for w in wts],
        out_specs=[
            pl.BlockSpec((1, TN, K, C2), lambda b, nt: (b, nt, 0, 0)),
            pl.BlockSpec((1, TN, K, C2), lambda b, nt: (b, nt, 0, 0)),
            pl.BlockSpec((1, TN, 3, C2), lambda b, nt: (b, nt, 0, 0)),
            pl.BlockSpec((1, 1, 2, C2), lambda b, nt: (b, nt, 0, 0)),
            pl.BlockSpec((1, 1, 2, C2), lambda b, nt: (b, nt, 0, 0)),
        ],
        out_shape=[
            jax.ShapeDtypeStruct((B, NP, K, C2), jnp.float32),
            jax.ShapeDtypeStruct((B, NP, K, C2), jnp.float32),
            jax.ShapeDtypeStruct((B, NP, 3, C2), jnp.float32),
            jax.ShapeDtypeStruct((B, NT, 2, C2), jnp.float32),
            jax.ShapeDtypeStruct((B, NT, 2, C2), jnp.float32),
        ],
    )(xcl, feat, *wts)


def _m2_body(o1_ref, g1_ref, ao1_ref, do1_ref, ag1_ref, dg1_ref,
             lw2T, lb2, W2T, b2, o2_ref, g2_ref, so2_ref, sg2_ref):
    nt = pl.program_id(1)
    msk = _nmask(nt, TN)
    a = ao1_ref[0]
    d = do1_ref[0]
    h = jnp.maximum(o1_ref[0] * a[None] + d[None], 0.0)
    o2 = jnp.dot(h.reshape(TN * K, C2), lw2T[...],
                 preferred_element_type=jnp.float32) + lb2[...]
    o2 = o2.reshape(TN, K, C2) * msk[:, None]
    o2_ref[0] = o2
    so2_ref[0, 0] = jnp.stack(
        [jnp.sum(o2, axis=(0, 1)), jnp.sum(o2 * o2, axis=(0, 1))])

    ag = ag1_ref[0]
    dg = dg1_ref[0]
    acc = b2[...] * jnp.ones((TN, 1), jnp.float32)
    for j in range(3):
        hg = jnp.maximum(g1_ref[0, :, j] * ag + dg, 0.0)
        acc = acc + jnp.dot(hg, W2T[j], preferred_element_type=jnp.float32)
    acc = acc * msk
    g2_ref[0] = acc
    sg2_ref[0, 0] = jnp.stack(
        [jnp.sum(acc, axis=0), jnp.sum(acc * acc, axis=0)])


def _m2(o1, g1, ao1, do1, ag1, dg1, wts):
    return pl.pallas_call(
        _m2_body,
        grid=(B, NT),
        in_specs=[
            pl.BlockSpec((1, TN, K, C2), lambda b, nt: (b, nt, 0, 0)),
            pl.BlockSpec((1, TN, 3, C2), lambda b, nt: (b, nt, 0, 0)),
            pl.BlockSpec((1, 1, C2), lambda b, nt: (b, 0, 0)),
            pl.BlockSpec((1, 1, C2), lambda b, nt: (b, 0, 0)),
            pl.BlockSpec((1, 1, C2), lambda b, nt: (0, 0, 0)),
            pl.BlockSpec((1, 1, C2), lambda b, nt: (0, 0, 0)),
        ] + [_wspec(w) for w in wts],
        out_specs=[
            pl.BlockSpec((1, TN, K, C2), lambda b, nt: (b, nt, 0, 0)),
            pl.BlockSpec((1, TN, C2), lambda b, nt: (b, nt, 0)),
            pl.BlockSpec((1, 1, 2, C2), lambda b, nt: (b, nt, 0, 0)),
            pl.BlockSpec((1, 1, 2, C2), lambda b, nt: (b, nt, 0, 0)),
        ],
        out_shape=[
            jax.ShapeDtypeStruct((B, NP, K, C2), jnp.float32),
            jax.ShapeDtypeStruct((B, NP, C2), jnp.float32),
            jax.ShapeDtypeStruct((B, NT, 2, C2), jnp.float32),
            jax.ShapeDtypeStruct((B, NT, 2, C2), jnp.float32),
        ],
    )(o1, g1, ao1, do1, ag1, dg1, *wts)


def _m3_body(o2_ref, x1_ref, g2_ref, ao2_ref, do2_ref, ag2_ref, dg2_ref,
             rwT2, rb2, lw1T2, lb12, rwT1, rb1, lw1T1, lb11,
             x1m_ref, om1_ref, x1a_ref, oa1_ref, som_ref, soa_ref):
    nt = pl.program_id(1)
    msk = _nmask(nt, TN)
    a = ao2_ref[0]
    d = do2_ref[0]
    r = jnp.maximum(o2_ref[0] * a[None] + d[None] + x1_ref[0], 0.0)
    mx0 = jnp.max(r, axis=1) * msk
    x1m = (jnp.dot(mx0, rwT2[...], preferred_element_type=jnp.float32)
           + rb2[...]) * msk
    om1 = (jnp.dot(mx0, lw1T2[...], preferred_element_type=jnp.float32)
           + lb12[...]) * msk
    x1m_ref[0] = x1m
    om1_ref[0] = om1
    som_ref[0, 0] = jnp.stack(
        [jnp.sum(om1, axis=0), jnp.sum(om1 * om1, axis=0)])

    an0 = jnp.maximum(g2_ref[0] * ag2_ref[0] + dg2_ref[0], 0.0) * msk
    x1a = (jnp.dot(an0, rwT1[...], preferred_element_type=jnp.float32)
           + rb1[...]) * msk
    oa1 = (jnp.dot(an0, lw1T1[...], preferred_element_type=jnp.float32)
           + lb11[...]) * msk
    x1a_ref[0] = x1a
    oa1_ref[0] = oa1
    soa_ref[0, 0] = jnp.stack(
        [jnp.sum(oa1, axis=0), jnp.sum(oa1 * oa1, axis=0)])


def _m3(o2, x1, g2, ao2, do2, ag2, dg2, wts):
    return pl.pallas_call(
        _m3_body,
        grid=(B, NT),
        in_specs=[
            pl.BlockSpec((1, TN, K, C2), lambda b, nt: (b, nt, 0, 0)),
            pl.BlockSpec((1, TN, K, C2), lambda b, nt: (b, nt, 0, 0)),
            pl.BlockSpec((1, TN, C2), lambda b, nt: (b, nt, 0)),
            pl.BlockSpec((1, 1, C2), lambda b, nt: (b, 0, 0)),
            pl.BlockSpec((1, 1, C2), lambda b, nt: (b, 0, 0)),
            pl.BlockSpec((1, 1, C2), lambda b, nt: (0, 0, 0)),
            pl.BlockSpec((1, 1, C2), lambda b, nt: (0, 0, 0)),
        ] + [_wspec(w) for w in wts],
        out_specs=[
            pl.BlockSpec((1, TN, C), lambda b, nt: (b, nt, 0)),
            pl.BlockSpec((1, TN, C), lambda b, nt: (b, nt, 0)),
            pl.BlockSpec((1, TN, C), lambda b, nt: (b, nt, 0)),
            pl.BlockSpec((1, TN, C), lambda b, nt: (b, nt, 0)),
            pl.BlockSpec((1, 1, 2, C), lambda b, nt: (b, nt, 0, 0)),
            pl.BlockSpec((1, 1, 2, C), lambda b, nt: (b, nt, 0, 0)),
        ],
        out_shape=[
            jax.ShapeDtypeStruct((B, NP, C), jnp.float32),
            jax.ShapeDtypeStruct((B, NP, C), jnp.float32),
            jax.ShapeDtypeStruct((B, NP, C), jnp.float32),
            jax.ShapeDtypeStruct((B, NP, C), jnp.float32),
            jax.ShapeDtypeStruct((B, NT, 2, C), jnp.float32),
            jax.ShapeDtypeStruct((B, NT, 2, C), jnp.float32),
        ],
    )(o2, x1, g2, ao2, do2, ag2, dg2, *wts)


def _m4_body(oa1_ref, om1_ref, aa_ref, da_ref, am_ref, dm_ref,
             lw2T1, lb21, lw2T2, lb22, oa2_ref, om2_ref, sa_ref, sm_ref):
    nt = pl.program_id(1)
    msk = _nmask(nt, TN)
    ha = jnp.maximum(oa1_ref[0] * aa_ref[0] + da_ref[0], 0.0)
    oa2 = (jnp.dot(ha, lw2T1[...], preferred_element_type=jnp.float32)
           + lb21[...]) * msk
    oa2_ref[0] = oa2
    sa_ref[0, 0] = jnp.stack(
        [jnp.sum(oa2, axis=0), jnp.sum(oa2 * oa2, axis=0)])
    hm = jnp.maximum(om1_ref[0] * am_ref[0] + dm_ref[0], 0.0)
    om2 = (jnp.dot(hm, lw2T2[...], preferred_element_type=jnp.float32)
           + lb22[...]) * msk
    om2_ref[0] = om2
    sm_ref[0, 0] = jnp.stack(
        [jnp.sum(om2, axis=0), jnp.sum(om2 * om2, axis=0)])


def _m4(oa1, om1, aa, da, am, dm, wts):
    return pl.pallas_call(
        _m4_body,
        grid=(B, NT),
        in_specs=[
            pl.BlockSpec((1, TN, C), lambda b, nt: (b, nt, 0)),
            pl.BlockSpec((1, TN, C), lambda b, nt: (b, nt, 0)),
            pl.BlockSpec((1, 1, C), lambda b, nt: (b, 0, 0)),
            pl.BlockSpec((1, 1, C), lambda b, nt: (b, 0, 0)),
            pl.BlockSpec((1, 1, C), lambda b, nt: (b, 0, 0)),
            pl.BlockSpec((1, 1, C), lambda b, nt: (b, 0, 0)),
        ] + [_wspec(w) for w in wts],
        out_specs=[
            pl.BlockSpec((1, TN, C), lambda b, nt: (b, nt, 0)),
            pl.BlockSpec((1, TN, C), lambda b, nt: (b, nt, 0)),
            pl.BlockSpec((1, 1, 2, C), lambda b, nt: (b, nt, 0, 0)),
            pl.BlockSpec((1, 1, 2, C), lambda b, nt: (b, nt, 0, 0)),
        ],
        out_shape=[
            jax.ShapeDtypeStruct((B, NP, C), jnp.float32),
            jax.ShapeDtypeStruct((B, NP, C), jnp.float32),
            jax.ShapeDtypeStruct((B, NT, 2, C), jnp.float32),
            jax.ShapeDtypeStruct((B, NT, 2, C), jnp.float32),
        ],
    )(oa1, om1, aa, da, am, dm, *wts)


def _m5_body(oa2_ref, x1a_ref, om2_ref, x1m_ref, aa_ref, da_ref,
             am_ref, dm_ref, an_ref, mx_ref, sa_ref, sm_ref):
    nt = pl.program_id(1)
    msk = _nmask(nt, TN)
    an = jnp.maximum(oa2_ref[0] * aa_ref[0] + da_ref[0] + x1a_ref[0],
                     0.0) * msk
    mx = jnp.maximum(om2_ref[0] * am_ref[0] + dm_ref[0] + x1m_ref[0],
                     0.0) * msk
    an_ref[0] = an
    mx_ref[0] = mx
    sa_ref[0, 0] = jnp.stack([jnp.sum(an, axis=0), jnp.sum(an * an, axis=0)])
    sm_ref[0, 0] = jnp.stack([jnp.sum(mx, axis=0), jnp.sum(mx * mx, axis=0)])


def _m5(oa2, x1a, om2, x1m, aa, da, am, dm):
    return pl.pallas_call(
        _m5_body,
        grid=(B, NT),
        in_specs=[
            pl.BlockSpec((1, TN, C), lambda b, nt: (b, nt, 0)),
            pl.BlockSpec((1, TN, C), lambda b, nt: (b, nt, 0)),
            pl.BlockSpec((1, TN, C), lambda b, nt: (b, nt, 0)),
            pl.BlockSpec((1, TN, C), lambda b, nt: (b, nt, 0)),
            pl.BlockSpec((1, 1, C), lambda b, nt: (b, 0, 0)),
            pl.BlockSpec((1, 1, C), lambda b, nt: (b, 0, 0)),
            pl.BlockSpec((1, 1, C), lambda b, nt: (b, 0, 0)),
            pl.BlockSpec((1, 1, C), lambda b, nt: (b, 0, 0)),
        ],
        out_specs=[
            pl.BlockSpec((1, TN, C), lambda b, nt: (b, nt, 0)),
            pl.BlockSpec((1, TN, C), lambda b, nt: (b, nt, 0)),
            pl.BlockSpec((1, 1, 2, C), lambda b, nt: (b, nt, 0, 0)),
            pl.BlockSpec((1, 1, 2, C), lambda b, nt: (b, nt, 0, 0)),
        ],
        out_shape=[
            jax.ShapeDtypeStruct((B, NP, C), jnp.float32),
            jax.ShapeDtypeStruct((B, NP, C), jnp.float32),
            jax.ShapeDtypeStruct((B, NT, 2, C), jnp.float32),
            jax.ShapeDtypeStruct((B, NT, 2, C), jnp.float32),
        ],
    )(oa2, x1a, om2, x1m, aa, da, am, dm)


def _m6_body(an_ref, mx_ref, aa_ref, da_ref, am_ref, dm_ref,
             cg1T, cb1, cg2T, cb2, cg3T, cb3, cg4T, cb4,
             l1_ref, l2_ref, s3_ref, s4_ref, m1_ref, m2_ref):
    nt = pl.program_id(1)
    msk = _nmask(nt, TN)
    neg = (1.0 - msk) * (-1e30)
    hn_an = jnp.maximum(an_ref[0] * aa_ref[0] + da_ref[0], 0.0)
    hn_mx = jnp.maximum(mx_ref[0] * am_ref[0] + dm_ref[0], 0.0)
    l1 = (jnp.dot(hn_an, cg1T[...], preferred_element_type=jnp.float32)
          + cb1[...]) * msk + neg
    l2 = (jnp.dot(hn_mx, cg2T[...], preferred_element_type=jnp.float32)
          + cb2[...]) * msk + neg
    l1_ref[0] = l1
    l2_ref[0] = l2
    m1_ref[0, 0, 0] = jnp.max(l1, axis=0)
    m2_ref[0, 0, 0] = jnp.max(l2, axis=0)

    l3 = (jnp.dot(hn_mx, cg3T[...], preferred_element_type=jnp.float32)
          + cb3[...])
    l4 = (jnp.dot(hn_an, cg4T[...], preferred_element_type=jnp.float32)
          + cb4[...])
    e3 = jnp.exp(l3 - jnp.max(l3, axis=1, keepdims=True))
    s3_ref[0] = (e3 / jnp.sum(e3, axis=1, keepdims=True)) * msk
    e4 = jnp.exp(l4 - jnp.max(l4, axis=1, keepdims=True))
    s4_ref[0] = (e4 / jnp.sum(e4, axis=1, keepdims=True)) * msk


def _m6(an, mx, aa, da, am, dm, cgw):
    return pl.pallas_call(
        _m6_body,
        grid=(B, NT),
        in_specs=[
            pl.BlockSpec((1, TN, C), lambda b, nt: (b, nt, 0)),
            pl.BlockSpec((1, TN, C), lambda b, nt: (b, nt, 0)),
            pl.BlockSpec((1, 1, C), lambda b, nt: (b, 0, 0)),
            pl.BlockSpec((1, 1, C), lambda b, nt: (b, 0, 0)),
            pl.BlockSpec((1, 1, C), lambda b, nt: (b, 0, 0)),
            pl.BlockSpec((1, 1, C), lambda b, nt: (b, 0, 0)),
        ] + [_wspec(w) for w in cgw],
        out_specs=[
            pl.BlockSpec((1, TN, CL), lambda b, nt: (b, nt, 0)),
            pl.BlockSpec((1, TN, CL), lambda b, nt: (b, nt, 0)),
            pl.BlockSpec((1, TN, CL), lambda b, nt: (b, nt, 0)),
            pl.BlockSpec((1, TN, CL), lambda b, nt: (b, nt, 0)),
            pl.BlockSpec((1, 1, 1, CL), lambda b, nt: (b, nt, 0, 0)),
            pl.BlockSpec((1, 1, 1, CL), lambda b, nt: (b, nt, 0, 0)),
        ],
        out_shape=[
            jax.ShapeDtypeStruct((B, NP, CL), jnp.float32),
            jax.ShapeDtypeStruct((B, NP, CL), jnp.float32),
            jax.ShapeDtypeStruct((B, NP, CL), jnp.float32),
            jax.ShapeDtypeStruct((B, NP, CL), jnp.float32),
            jax.ShapeDtypeStruct((B, NT, 1, CL), jnp.float32),
            jax.ShapeDtypeStruct((B, NT, 1, CL), jnp.float32),
        ],
    )(an, mx, aa, da, am, dm, *cgw)


def _m7a_body(l1_ref, l2_ref, mm1_ref, mm2_ref, es1_ref, es2_ref):
    e1 = jnp.exp(l1_ref[0] - mm1_ref[0])
    e2 = jnp.exp(l2_ref[0] - mm2_ref[0])
    es1_ref[0, 0, 0] = jnp.sum(e1, axis=0)
    es2_ref[0, 0, 0] = jnp.sum(e2, axis=0)


def _m7a(l1, l2, mm1, mm2):
    return pl.pallas_call(
        _m7a_body,
        grid=(B, NT),
        in_specs=[
            pl.BlockSpec((1, TN, CL), lambda b, nt: (b, nt, 0)),
            pl.BlockSpec((1, TN, CL), lambda b, nt: (b, nt, 0)),
            pl.BlockSpec((1, 1, CL), lambda b, nt: (b, 0, 0)),
            pl.BlockSpec((1, 1, CL), lambda b, nt: (b, 0, 0)),
        ],
        out_specs=[
            pl.BlockSpec((1, 1, 1, CL), lambda b, nt: (b, nt, 0, 0)),
            pl.BlockSpec((1, 1, 1, CL), lambda b, nt: (b, nt, 0, 0)),
        ],
        out_shape=[
            jax.ShapeDtypeStruct((B, NT, 1, CL), jnp.float32),
            jax.ShapeDtypeStruct((B, NT, 1, CL), jnp.float32),
        ],
    )(l1, l2, mm1, mm2)


def _m7_body(l1_ref, l2_ref, an_ref, mx_ref, mm1_ref, mm2_ref,
             es1_ref, es2_ref, ca_ref, cm_ref):
    nt = pl.program_id(1)
    s1 = jnp.exp(l1_ref[0] - mm1_ref[0]) / es1_ref[0]
    s2 = jnp.exp(l2_ref[0] - mm2_ref[0]) / es2_ref[0]
    pa = lax.dot_general(s1, an_ref[0], (((0,), (0,)), ((), ())),
                         preferred_element_type=jnp.float32)
    pm = lax.dot_general(s2, mx_ref[0], (((0,), (0,)), ((), ())),
                         preferred_element_type=jnp.float32)

    @pl.when(nt == 0)
    def _():
        ca_ref[0] = pa
        cm_ref[0] = pm

    @pl.when(nt != 0)
    def _():
        ca_ref[0] += pa
        cm_ref[0] += pm


def _m7(l1, l2, an, mx, mm1, mm2, es1, es2):
    return pl.pallas_call(
        _m7_body,
        grid=(B, NT),
        in_specs=[
            pl.BlockSpec((1, TN, CL), lambda b, nt: (b, nt, 0)),
            pl.BlockSpec((1, TN, CL), lambda b, nt: (b, nt, 0)),
            pl.BlockSpec((1, TN, C), lambda b, nt: (b, nt, 0)),
            pl.BlockSpec((1, TN, C), lambda b, nt: (b, nt, 0)),
            pl.BlockSpec((1, 1, CL), lambda b, nt: (b, 0, 0)),
            pl.BlockSpec((1, 1, CL), lambda b, nt: (b, 0, 0)),
            pl.BlockSpec((1, 1, CL), lambda b, nt: (b, 0, 0)),
            pl.BlockSpec((1, 1, CL), lambda b, nt: (b, 0, 0)),
        ],
        out_specs=[
            pl.BlockSpec((1, CL, C), lambda b, nt: (b, 0, 0)),
            pl.BlockSpec((1, CL, C), lambda b, nt: (b, 0, 0)),
        ],
        out_shape=[
            jax.ShapeDtypeStruct((B, CL, C), jnp.float32),
            jax.ShapeDtypeStruct((B, CL, C), jnp.float32),
        ],
    )(l1, l2, an, mx, mm1, mm2, es1, es2)


def _bn_cl(y):
    m = jnp.mean(y, axis=(0, 1), keepdims=True)
    d = y - m
    v = jnp.mean(d * d, axis=(0, 1), keepdims=True)
    return d * lax.rsqrt(v + EPS)


def _sigmoid(x):
    return 1.0 / (1.0 + jnp.exp(-x))


def _c1_body(ca_ref, cm_ref, wref, fa_ref, fm_ref):
    iw = [wref[i * 9:(i + 1) * 9] for i in range(4)]
    aw1 = wref[36:44]
    aw2 = wref[44:52]

    ca = ca_ref[...]
    cm = cm_ref[...]

    def conv(x, wT, bb):
        outs = [jnp.dot(x[b], wT[...], preferred_element_type=jnp.float32)
                + bb[...] for b in range(B)]
        return jnp.stack(outs)

    def inter(n1, n2, n3, p):
        qwT, qb, kwT, kb, vwT, vb, cwT, cb, gamma = p
        q = jnp.maximum(_bn_cl(conv(n1, qwT, qb)), 0.0)
        kk = jnp.maximum(_bn_cl(conv(n2, kwT, kb)), 0.0)
        v = jnp.maximum(_bn_cl(conv(n3, vwT, vb)), 0.0)
        outs = []
        for b in range(B):
            sc = lax.dot_general(q[b], kk[b], (((1,), (1,)), ((), ())),
                                 preferred_element_type=jnp.float32)
            e = jnp.exp(sc - jnp.max(sc, axis=1, keepdims=True))
            att = e / jnp.sum(e, axis=1, keepdims=True)
            outs.append(jnp.dot(att, v[b],
                                preferred_element_type=jnp.float32))
        out = jnp.stack(outs)
        out = jnp.maximum(_bn_cl(conv(out, cwT, cb)), 0.0)
        return n3 + gamma[...] * out

    def aff_cl(x, res, p):
        lw1T, lb1, lw2T, lb2, gw1T, gb1, gw2T, gb2 = p
        xa = x + res
        xl = _bn_cl(conv(jnp.maximum(_bn_cl(conv(xa, lw1T, lb1)), 0.0),
                         lw2T, lb2))
        xg0 = jnp.mean(xa, axis=1, keepdims=True)
        g = conv(xg0, gw1T, gb1)
        gd = g - jnp.mean(g, axis=0, keepdims=True)
        gv = jnp.mean(gd * gd, axis=0, keepdims=True)
        g = jnp.maximum(gd * lax.rsqrt(gv + EPS), 0.0)
        g = conv(g, gw2T, gb2)
        gd = g - jnp.mean(g, axis=0, keepdims=True)
        gv = jnp.mean(gd * gd, axis=0, keepdims=True)
        xg = gd * lax.rsqrt(gv + EPS)
        wei = _sigmoid(xl + xg)
        return 2.0 * x * wei + 2.0 * res * (1.0 - wei)

    an_inter = inter(ca, ca, ca, iw[0])
    max_inter = inter(cm, cm, cm, iw[1])
    an_intra = inter(ca, cm, cm, iw[2])
    max_intra = inter(cm, ca, ca, iw[3])
    fa_ref[...] = aff_cl(an_inter, an_intra, aw1)
    fm_ref[...] = aff_cl(max_inter, max_intra, aw2)


def _c1(ca, cm, wts):
    n_w = len(wts)

    def body(*refs):
        _c1_body(refs[0], refs[1], refs[2:2 + n_w],
                 refs[2 + n_w], refs[3 + n_w])

    return pl.pallas_call(
        body,
        out_shape=[
            jax.ShapeDtypeStruct((B, CL, C), jnp.float32),
            jax.ShapeDtypeStruct((B, CL, C), jnp.float32),
        ],
    )(ca, cm, *wts)


def _m8_body(s3_ref, s4_ref, fa_ref, fm_ref, lw1T, lb1,
             om2_ref, oa2_ref, l1_ref, sl1_ref, sxa_ref):
    nt = pl.program_id(1)
    msk = _nmask(nt, TN)
    om2 = jnp.dot(s3_ref[0], fa_ref[0], preferred_element_type=jnp.float32)
    oa2 = jnp.dot(s4_ref[0], fm_ref[0], preferred_element_type=jnp.float32)
    om2_ref[0] = om2
    oa2_ref[0] = oa2
    xa = om2 + oa2
    l1 = (jnp.dot(xa, lw1T[...], preferred_element_type=jnp.float32)
          + lb1[...]) * msk
    l1_ref[0] = l1
    sl1_ref[0, 0] = jnp.stack([jnp.sum(l1, axis=0), jnp.sum(l1 * l1, axis=0)])
    sxa_ref[0, 0, 0] = jnp.sum(xa * msk, axis=0)


def _m8(s3, s4, fa, fm, lw1T, lb1):
    return pl.pallas_call(
        _m8_body,
        grid=(B, NT),
        in_specs=[
            pl.BlockSpec((1, TN, CL), lambda b, nt: (b, nt, 0)),
            pl.BlockSpec((1, TN, CL), lambda b, nt: (b, nt, 0)),
            pl.BlockSpec((1, CL, C), lambda b, nt: (b, 0, 0)),
            pl.BlockSpec((1, CL, C), lambda b, nt: (b, 0, 0)),
            pl.BlockSpec((C, C4), lambda b, nt: (0, 0)),
            pl.BlockSpec((1, C4), lambda b, nt: (0, 0)),
        ],
        out_specs=[
            pl.BlockSpec((1, TN, C), lambda b, nt: (b, nt, 0)),
            pl.BlockSpec((1, TN, C), lambda b, nt: (b, nt, 0)),
            pl.BlockSpec((1, TN, C4), lambda b, nt: (b, nt, 0)),
            pl.BlockSpec((1, 1, 2, C4), lambda b, nt: (b, nt, 0, 0)),
            pl.BlockSpec((1, 1, 1, C), lambda b, nt: (b, nt, 0, 0)),
        ],
        out_shape=[
            jax.ShapeDtypeStruct((B, NP, C), jnp.float32),
            jax.ShapeDtypeStruct((B, NP, C), jnp.float32),
            jax.ShapeDtypeStruct((B, NP, C4), jnp.float32),
            jax.ShapeDtypeStruct((B, NT, 2, C4), jnp.float32),
            jax.ShapeDtypeStruct((B, NT, 1, C), jnp.float32),
        ],
    )(s3, s4, fa, fm, lw1T, lb1)


def _m9_body(l1_ref, a_ref, d_ref, lw2T, lb2, l2_ref, sl2_ref):
    nt = pl.program_id(1)
    msk = _nmask(nt, TN)
    h = jnp.maximum(l1_ref[0] * a_ref[...] + d_ref[...], 0.0)
    l2 = (jnp.dot(h, lw2T[...], preferred_element_type=jnp.float32)
          + lb2[...]) * msk
    l2_ref[0] = l2
    sl2_ref[0, 0] = jnp.stack([jnp.sum(l2, axis=0), jnp.sum(l2 * l2, axis=0)])


def _m9(l1, a, d, lw2T, lb2):
    return pl.pallas_call(
        _m9_body,
        grid=(B, NT),
        in_specs=[
            pl.BlockSpec((1, TN, C4), lambda b, nt: (b, nt, 0)),
            pl.BlockSpec((1, C4), lambda b, nt: (0, 0)),
            pl.BlockSpec((1, C4), lambda b, nt: (0, 0)),
            pl.BlockSpec((C4, C), lambda b, nt: (0, 0)),
            pl.BlockSpec((1, C), lambda b, nt: (0, 0)),
        ],
        out_specs=[
            pl.BlockSpec((1, TN, C), lambda b, nt: (b, nt, 0)),
            pl.BlockSpec((1, 1, 2, C), lambda b, nt: (b, nt, 0, 0)),
        ],
        out_shape=[
            jax.ShapeDtypeStruct((B, NP, C), jnp.float32),
            jax.ShapeDtypeStruct((B, NT, 2, C), jnp.float32),
        ],
    )(l1, a, d, lw2T, lb2)


def _m10_body(l2_ref, om2_ref, oa2_ref, a_ref, d_ref, xg0_ref,
              gw1T, gb1, gw2T, gb2, out_ref):
    b = pl.program_id(0)
    g = (jnp.dot(xg0_ref[...], gw1T[...],
                 preferred_element_type=jnp.float32) + gb1[...])
    gd = g - jnp.mean(g, axis=0, keepdims=True)
    gv = jnp.mean(gd * gd, axis=0, keepdims=True)
    g = jnp.maximum(gd * lax.rsqrt(gv + EPS), 0.0)
    g = (jnp.dot(g, gw2T[...], preferred_element_type=jnp.float32)
         + gb2[...])
    gd = g - jnp.mean(g, axis=0, keepdims=True)
    gv = jnp.mean(gd * gd, axis=0, keepdims=True)
    xg = gd * lax.rsqrt(gv + EPS)
    bio = lax.broadcasted_iota(jnp.int32, (B, 1), 0)
    xgb = jnp.sum(jnp.where(bio == b, xg, 0.0), axis=0, keepdims=True)
    xl = l2_ref[0] * a_ref[...] + d_ref[...]
    wei = _sigmoid(xl + xgb)
    out_ref[0] = 2.0 * om2_ref[0] * wei + 2.0 * oa2_ref[0] * (1.0 - wei)


def _m10(l2, om2, oa2, a, d, xg0, gw1T, gb1, gw2T, gb2):
    return pl.pallas_call(
        _m10_body,
        grid=(B, NT),
        in_specs=[
            pl.BlockSpec((1, TN, C), lambda b, nt: (b, nt, 0)),
            pl.BlockSpec((1, TN, C), lambda b, nt: (b, nt, 0)),
            pl.BlockSpec((1, TN, C), lambda b, nt: (b, nt, 0)),
            pl.BlockSpec((1, C), lambda b, nt: (0, 0)),
            pl.BlockSpec((1, C), lambda b, nt: (0, 0)),
            pl.BlockSpec((B, C), lambda b, nt: (0, 0)),
            pl.BlockSpec((C, C4), lambda b, nt: (0, 0)),
            pl.BlockSpec((1, C4), lambda b, nt: (0, 0)),
            pl.BlockSpec((C4, C), lambda b, nt: (0, 0)),
            pl.BlockSpec((1, C), lambda b, nt: (0, 0)),
        ],
        out_specs=pl.BlockSpec((1, TN, C), lambda b, nt: (b, nt, 0)),
        out_shape=jax.ShapeDtypeStruct((B, NP, C), jnp.float32),
    )(l2, om2, oa2, a, d, xg0, gw1T, gb1, gw2T, gb2)


def _inst_bn_scale(stats, cnt, eps_i=EPS):
    s = jnp.sum(stats[:, :, 0, :], axis=1)
    q = jnp.sum(stats[:, :, 1, :], axis=1)
    m = s / cnt
    v = q / cnt - m * m
    inv = lax.rsqrt(v + eps_i)
    Vc = jnp.mean(v / (v + eps_i), axis=0, keepdims=True)
    a = inv * lax.rsqrt(Vc + EPS)
    d = -m * a
    return a[:, None, :], d[:, None, :]


def _bn_scale(stats, cnt):
    s = jnp.sum(stats[:, :, 0, :], axis=(0, 1))
    q = jnp.sum(stats[:, :, 1, :], axis=(0, 1))
    m = s / cnt
    v = q / cnt - m * m
    a = lax.rsqrt(v + EPS)
    return a[None, :], (-m * a)[None, :]


def _r2(b):
    return b.reshape(1, -1)


def kernel(features, params):
    p = params
    x = features[:, :, :, 0]
    xx = jnp.pad(jnp.sum(x * x, axis=1), ((0, 0), (0, NP - N)))
    xcf = jnp.pad(x, ((0, 0), (0, 0), (0, NP - N)))
    xcl = jnp.transpose(xcf, (0, 2, 1))

    idxg = _knn(xcl, xcf, xx)
    feat = _gather_feat(xcl.reshape(B * NP, C),
                        idxg.reshape(_NROWS)).reshape(B, NP, K, C)

    w1, b1 = p['conv']['w1'], p['conv']['b1']
    w2, b2 = p['conv']['w2'], p['conv']['b2']
    W1T = jnp.transpose(w1, (2, 1, 0))
    W2T = jnp.transpose(w2, (2, 1, 0))
    m1p = p['mlp1']

    o1, x1, g1, so1, sg1 = _m1(
        xcl, feat, (m1p['lw1'].T, _r2(m1p['lb1']), m1p['rw'].T,
                    _r2(m1p['rb']), W1T, _r2(b1)))
    ao1, do1 = _inst_bn_scale(so1, K * N)
    ag1, dg1 = _bn_scale(sg1, B * 3 * N)

    o2, g2, so2, sg2 = _m2(o1, g1, ao1, do1, ag1[None], dg1[None],
                           (m1p['lw2'].T, _r2(m1p['lb2']), W2T, _r2(b2)))
    ao2, do2 = _inst_bn_scale(so2, K * N)
    ag2, dg2 = _bn_scale(sg2, B * N)

    c1p, c2p = p['change1'], p['change2']
    x1m, om1, x1a, oa1, som, soa = _m3(
        o2, x1, g2, ao2, do2, ag2[None], dg2[None],
        (c2p['rw'].T, _r2(c2p['rb']), c2p['lw1'].T, _r2(c2p['lb1']),
         c1p['rw'].T, _r2(c1p['rb']), c1p['lw1'].T, _r2(c1p['lb1'])))
    aoa1, doa1 = _inst_bn_scale(soa, N)
    aom1, dom1 = _inst_bn_scale(som, N)

    oa2, om2, soa2, som2 = _m4(oa1, om1, aoa1, doa1, aom1, dom1,
                               (c1p['lw2'].T, _r2(c1p['lb2']),
                                c2p['lw2'].T, _r2(c2p['lb2'])))
    aoa2, doa2 = _inst_bn_scale(soa2, N)
    aom2, dom2 = _inst_bn_scale(som2, N)

    an, mx, san, smx = _m5(oa2, x1a, om2, x1m, aoa2, doa2, aom2, dom2)
    aan, dan = _inst_bn_scale(san, N, eps_i=1e-3)
    amx, dmx = _inst_bn_scale(smx, N, eps_i=1e-3)

    cgw = (p['cg1']['w'].T, _r2(p['cg1']['b']),
           p['cg2']['w'].T, _r2(p['cg2']['b']),
           p['cg3']['w'].T, _r2(p['cg3']['b']),
           p['cg4']['w'].T, _r2(p['cg4']['b']))
    l1, l2, s3, s4, pm1, pm2 = _m6(an, mx, aan, dan, amx, dmx, cgw)
    mm1 = jnp.max(pm1[:, :, 0, :], axis=1, keepdims=True)
    mm2 = jnp.max(pm2[:, :, 0, :], axis=1, keepdims=True)

    es1p, es2p = _m7a(l1, l2, mm1, mm2)
    es1 = jnp.sum(es1p[:, :, 0, :], axis=1)[:, None, :]
    es2 = jnp.sum(es2p[:, :, 0, :], axis=1)[:, None, :]
    ca, cm = _m7(l1, l2, an, mx, mm1, mm2, es1, es2)

    iw = []
    for nm in ['inter1', 'inter2', 'intra1', 'intra2']:
        q = p[nm]
        iw += [q['qw'].T, _r2(q['qb']), q['kw'].T, _r2(q['kb']),
               q['vw'].T, _r2(q['vb']), q['cw'].T, _r2(q['cb']),
               _r2(q['gamma'])]
    for nm in ['aff1', 'aff2']:
        q = p[nm]
        iw += [q['lw1'].T, _r2(q['lb1']), q['lw2'].T, _r2(q['lb2']),
               q['gw1'].T, _r2(q['gb1']), q['gw2'].T, _r2(q['gb2'])]
    fa, fm = _c1(ca, cm, iw)

    a3 = p['aff3']
    om2p, oa2p, al1, sl1, sxa = _m8(s3, s4, fa, fm, a3['lw1'].T,
                                    _r2(a3['lb1']))
    aal1, dal1 = _bn_scale(sl1, B * N)
    xg0 = jnp.sum(sxa[:, :, 0, :], axis=1) / N

    al2, sl2 = _m9(al1, aal1, dal1, a3['lw2'].T, _r2(a3['lb2']))
    aal2, dal2 = _bn_scale(sl2, B * N)

    out = _m10(al2, om2p, oa2p, aal2, dal2, xg0,
               a3['gw1'].T, _r2(a3['gb1']), a3['gw2'].T, _r2(a3['gb2']))
    return jnp.transpose(out[:, :N, :], (0, 2, 1))[:, :, :, None]
